# baseline (speedup 1.0000x reference)
import functools

import jax
import jax.numpy as jnp
from jax import lax
from jax.experimental import pallas as pl
from jax.experimental.pallas import tpu as pltpu
from jax.experimental.pallas import tpu_sc as plsc

B = 32
CH = 32
HW = 224 * 224
NCLS = 19
CLS_PAD = 20
ACC = 2 * CH * CLS_PAD
L = 16


def _sc_body(src_fea, trg_fea, src_lab, trg_lab, out_s, out_c,
             lab_buf, fea_buf, sum_acc, cnt_acc):
    nc = 2
    wid = lax.axis_index("s") * nc + lax.axis_index("c")
    b = wid

    zeros = jnp.zeros((L,), jnp.float32)
    ones = jnp.full((L,), 1.0, jnp.float32)

    def zero_body(i, c):
        sum_acc[pl.ds(i * L, L)] = zeros
        cnt_acc[pl.ds(i * L, L)] = zeros
        return c
    lax.fori_loop(0, ACC // L, zero_body, 0)

    for tensor, (fea_hbm, lab_hbm) in enumerate(
            ((src_fea, src_lab), (trg_fea, trg_lab))):
        pltpu.sync_copy(lab_hbm.at[pl.ds(b * HW, HW)], lab_buf)

        cnt_base = (tensor * B + b) * CLS_PAD

        def cnt_body(p, c):
            l16 = lab_buf[pl.ds(p * L, L)]
            plsc.addupdate_scatter(cnt_acc, [l16 + cnt_base], ones)
            return c
        lax.fori_loop(0, HW // L, cnt_body, 0)

        sum_base_t = tensor * CH * CLS_PAD

        def ch_body(ch, c):
            pltpu.sync_copy(fea_hbm.at[pl.ds((b * CH + ch) * HW, HW)], fea_buf)
            base = sum_base_t + ch * CLS_PAD

            def px_body(p, c2):
                l16 = lab_buf[pl.ds(p * L, L)]
                v16 = fea_buf[pl.ds(p * L, L)]
                plsc.addupdate_scatter(sum_acc, [l16 + base], v16)
                return c2
            lax.fori_loop(0, HW // L, px_body, 0)
            return c
        lax.fori_loop(0, CH, ch_body, 0)

    pltpu.sync_copy(sum_acc, out_s.at[wid])
    pltpu.sync_copy(cnt_acc, out_c.at[wid])


_sc_seg = functools.partial(
    pl.kernel,
    mesh=plsc.VectorSubcoreMesh(core_axis_name="c", subcore_axis_name="s"),
    out_type=[
        jax.ShapeDtypeStruct((B, ACC), jnp.float32),
        jax.ShapeDtypeStruct((B, ACC), jnp.float32),
    ],
    scratch_types=[
        pltpu.VMEM((HW,), jnp.int32),
        pltpu.VMEM((HW,), jnp.float32),
        pltpu.VMEM((ACC,), jnp.float32),
        pltpu.VMEM((ACC,), jnp.float32),
    ],
    compiler_params=pltpu.CompilerParams(needs_layout_passes=False),
)(_sc_body)


def _loss_body(s_ref, c_ref, o_ref):
    s = jnp.sum(s_ref[...], axis=0, keepdims=True)
    n = jnp.sum(c_ref[...], axis=0, keepdims=True)
    ctr = s / ((n + 1e-8) * B)
    half = CH * CLS_PAD
    d = ctr[:, :half] - ctr[:, half:]
    o_ref[0, 0] = jnp.sqrt(jnp.sum(d * d))


def kernel(src_fea, trg_fea, src_labels, trg_pseudo_labels):
    sf = src_fea.reshape(-1)
    tf = trg_fea.reshape(-1)
    sl = src_labels.reshape(-1).astype(jnp.int32)
    tl = trg_pseudo_labels.reshape(-1).astype(jnp.int32)

    out_s, out_c = _sc_seg(sf, tf, sl, tl)

    loss = pl.pallas_call(
        _loss_body,
        out_shape=jax.ShapeDtypeStruct((1, 1), jnp.float32),
        out_specs=pl.BlockSpec(memory_space=pltpu.SMEM),
    )(out_s, out_c)
    return loss[0, 0]


# speedup vs baseline: 1.0539x; 1.0539x over previous
import functools

import jax
import jax.numpy as jnp
from jax import lax
from jax.experimental import pallas as pl
from jax.experimental.pallas import tpu as pltpu
from jax.experimental.pallas import tpu_sc as plsc

B = 32
CH = 32
HW = 224 * 224
NCLS = 19
CLS_PAD = 20
ACC = 2 * CH * CLS_PAD
L = 16
U = 8
NCHUNK = 8
P = HW // NCHUNK
NU = CH * NCHUNK


def _sc_body(src_fea, trg_fea, src_lab, trg_lab, out_s, out_c,
             lab_buf, fea_buf0, fea_buf1, sum_acc, cnt_acc, sem0, sem1):
    nc = 2
    wid = lax.axis_index("s") * nc + lax.axis_index("c")
    b = wid

    zeros = jnp.zeros((L,), jnp.float32)
    ones = jnp.full((L,), 1.0, jnp.float32)

    def zero_body(i, c):
        sum_acc[pl.ds(i * L, L)] = zeros
        cnt_acc[pl.ds(i * L, L)] = zeros
        return c
    lax.fori_loop(0, ACC // L, zero_body, 0)

    for tensor, (fea_hbm, lab_hbm) in enumerate(
            ((src_fea, src_lab), (trg_fea, trg_lab))):
        pltpu.sync_copy(lab_hbm.at[pl.ds(b * HW, HW)], lab_buf)

        sum_base_t = tensor * CH * CLS_PAD

        def _src_slice(u):
            ch = u // NCHUNK
            off = (u % NCHUNK) * P
            return fea_hbm.at[pl.ds((b * CH + ch) * HW + off, P)]

        def start_u(u, buf, sem):
            pltpu.async_copy(_src_slice(u), buf, sem)

        def wait_u(u, buf, sem):
            pltpu.make_async_copy(_src_slice(u), buf, sem).wait()

        def compute_u(u, buf):
            base = sum_base_t + (u // NCHUNK) * CLS_PAD
            lab_off = (u % NCHUNK) * P

            def px_body(p, c):
                o = p * (L * U)
                for j in range(U):
                    l16 = lab_buf[pl.ds(lab_off + o + j * L, L)]
                    v16 = buf[pl.ds(o + j * L, L)]
                    plsc.addupdate_scatter(sum_acc, [l16 + base], v16)
                return c
            lax.fori_loop(0, P // (L * U), px_body, 0)

        start_u(0, fea_buf0, sem0)

        cnt_base = (tensor * B + b) * CLS_PAD

        def cnt_body(p, c):
            o = p * (L * U)
            for j in range(U):
                l16 = lab_buf[pl.ds(o + j * L, L)]
                plsc.addupdate_scatter(cnt_acc, [l16 + cnt_base], ones)
            return c
        lax.fori_loop(0, HW // (L * U), cnt_body, 0)

        def pipe_body(u2, c):
            ua = 2 * u2
            start_u(ua + 1, fea_buf1, sem1)
            wait_u(ua, fea_buf0, sem0)
            compute_u(ua, fea_buf0)

            @pl.when(ua + 2 < NU)
            def _():
                start_u(ua + 2, fea_buf0, sem0)

            wait_u(ua + 1, fea_buf1, sem1)
            compute_u(ua + 1, fea_buf1)
            return c
        lax.fori_loop(0, NU // 2, pipe_body, 0)

    pltpu.sync_copy(sum_acc, out_s.at[wid])
    pltpu.sync_copy(cnt_acc, out_c.at[wid])


_sc_seg = functools.partial(
    pl.kernel,
    mesh=plsc.VectorSubcoreMesh(core_axis_name="c", subcore_axis_name="s"),
    out_type=[
        jax.ShapeDtypeStruct((B, ACC), jnp.float32),
        jax.ShapeDtypeStruct((B, ACC), jnp.float32),
    ],
    scratch_types=[
        pltpu.VMEM((HW,), jnp.int32),
        pltpu.VMEM((P,), jnp.float32),
        pltpu.VMEM((P,), jnp.float32),
        pltpu.VMEM((ACC,), jnp.float32),
        pltpu.VMEM((ACC,), jnp.float32),
        pltpu.SemaphoreType.DMA,
        pltpu.SemaphoreType.DMA,
    ],
    compiler_params=pltpu.CompilerParams(needs_layout_passes=False),
)(_sc_body)


def _loss_body(s_ref, c_ref, o_ref):
    s = jnp.sum(s_ref[...], axis=0, keepdims=True)
    n = jnp.sum(c_ref[...], axis=0, keepdims=True)
    ctr = s / ((n + 1e-8) * B)
    half = CH * CLS_PAD
    d = ctr[:, :half] - ctr[:, half:]
    o_ref[0, 0] = jnp.sqrt(jnp.sum(d * d))


def kernel(src_fea, trg_fea, src_labels, trg_pseudo_labels):
    sf = src_fea.reshape(-1)
    tf = trg_fea.reshape(-1)
    sl = src_labels.reshape(-1).astype(jnp.int32)
    tl = trg_pseudo_labels.reshape(-1).astype(jnp.int32)

    out_s, out_c = _sc_seg(sf, tf, sl, tl)

    loss = pl.pallas_call(
        _loss_body,
        out_shape=jax.ShapeDtypeStruct((1, 1), jnp.float32),
        out_specs=pl.BlockSpec(memory_space=pltpu.SMEM),
    )(out_s, out_c)
    return loss[0, 0]


# speedup vs baseline: 1.9569x; 1.8568x over previous
import functools

import jax
import jax.numpy as jnp
from jax import lax
from jax.experimental import pallas as pl
from jax.experimental.pallas import tpu as pltpu
from jax.experimental.pallas import tpu_sc as plsc

B = 32
CH = 32
HW = 224 * 224
NCLS = 19
CLS_PAD = 20
ACC = 2 * CH * CLS_PAD
L = 16
U = 8
NCHUNK = 8
P = HW // NCHUNK
NU = CH * NCHUNK


def _sc_body(src_fea, trg_fea, src_lab, trg_lab, out_s, out_c,
             lab_buf, fea_buf0, fea_buf1, sum_acc, cnt_acc, sem0, sem1):
    nc = 2
    wid = lax.axis_index("s") * nc + lax.axis_index("c")
    b = wid

    zeros = jnp.zeros((L,), jnp.float32)
    ones = jnp.full((L,), 1.0, jnp.float32)

    def zero_body(i, c):
        sum_acc[pl.ds(i * L, L)] = zeros
        cnt_acc[pl.ds(i * L, L)] = zeros
        return c
    lax.fori_loop(0, ACC // L, zero_body, 0)

    for tensor, (fea_hbm, lab_hbm) in enumerate(
            ((src_fea, src_lab), (trg_fea, trg_lab))):
        pltpu.sync_copy(lab_hbm.at[pl.ds(b * HW, HW)], lab_buf)

        sum_base_t = tensor * CH * CLS_PAD

        def _src_slice(u):
            ch = u // NCHUNK
            off = (u % NCHUNK) * P
            return fea_hbm.at[pl.ds((b * CH + ch) * HW + off, P)]

        def start_u(u, buf, sem):
            pltpu.async_copy(_src_slice(u), buf, sem)

        def wait_u(u, buf, sem):
            pltpu.make_async_copy(_src_slice(u), buf, sem).wait()

        def compute_u(u, buf):
            base = sum_base_t + (u // NCHUNK) * CLS_PAD
            lab_off = (u % NCHUNK) * P

            @plsc.parallel_loop(0, P // L, unroll=U)
            def _(p):
                o = p * L
                l16 = lab_buf[pl.ds(lab_off + o, L)]
                v16 = buf[pl.ds(o, L)]
                plsc.addupdate_scatter(sum_acc, [l16 + base], v16)

        start_u(0, fea_buf0, sem0)

        cnt_base = (tensor * B + b) * CLS_PAD

        @plsc.parallel_loop(0, HW // L, unroll=U)
        def _(p):
            l16 = lab_buf[pl.ds(p * L, L)]
            plsc.addupdate_scatter(cnt_acc, [l16 + cnt_base], ones)

        def pipe_body(u2, c):
            ua = 2 * u2
            start_u(ua + 1, fea_buf1, sem1)
            wait_u(ua, fea_buf0, sem0)
            compute_u(ua, fea_buf0)

            @pl.when(ua + 2 < NU)
            def _():
                start_u(ua + 2, fea_buf0, sem0)

            wait_u(ua + 1, fea_buf1, sem1)
            compute_u(ua + 1, fea_buf1)
            return c
        lax.fori_loop(0, NU // 2, pipe_body, 0)

    pltpu.sync_copy(sum_acc, out_s.at[wid])
    pltpu.sync_copy(cnt_acc, out_c.at[wid])


_sc_seg = functools.partial(
    pl.kernel,
    mesh=plsc.VectorSubcoreMesh(core_axis_name="c", subcore_axis_name="s"),
    out_type=[
        jax.ShapeDtypeStruct((B, ACC), jnp.float32),
        jax.ShapeDtypeStruct((B, ACC), jnp.float32),
    ],
    scratch_types=[
        pltpu.VMEM((HW,), jnp.int32),
        pltpu.VMEM((P,), jnp.float32),
        pltpu.VMEM((P,), jnp.float32),
        pltpu.VMEM((ACC,), jnp.float32),
        pltpu.VMEM((ACC,), jnp.float32),
        pltpu.SemaphoreType.DMA,
        pltpu.SemaphoreType.DMA,
    ],
    compiler_params=pltpu.CompilerParams(needs_layout_passes=False),
)(_sc_body)


def _loss_body(s_ref, c_ref, o_ref):
    s = jnp.sum(s_ref[...], axis=0, keepdims=True)
    n = jnp.sum(c_ref[...], axis=0, keepdims=True)
    ctr = s / ((n + 1e-8) * B)
    half = CH * CLS_PAD
    d = ctr[:, :half] - ctr[:, half:]
    o_ref[0, 0] = jnp.sqrt(jnp.sum(d * d))


def kernel(src_fea, trg_fea, src_labels, trg_pseudo_labels):
    sf = src_fea.reshape(-1)
    tf = trg_fea.reshape(-1)
    sl = src_labels.reshape(-1).astype(jnp.int32)
    tl = trg_pseudo_labels.reshape(-1).astype(jnp.int32)

    out_s, out_c = _sc_seg(sf, tf, sl, tl)

    loss = pl.pallas_call(
        _loss_body,
        out_shape=jax.ShapeDtypeStruct((1, 1), jnp.float32),
        out_specs=pl.BlockSpec(memory_space=pltpu.SMEM),
    )(out_s, out_c)
    return loss[0, 0]


# speedup vs baseline: 2.3406x; 1.1960x over previous
import functools

import jax
import jax.numpy as jnp
from jax import lax
from jax.experimental import pallas as pl
from jax.experimental.pallas import tpu as pltpu
from jax.experimental.pallas import tpu_sc as plsc

B = 32
CH = 32
HW = 224 * 224
NCLS = 19
CLS_PAD = 20
ACC = 2 * CH * CLS_PAD
L = 16
ACC_E = ACC * L
U = 8
NCHUNK = 8
P = HW // NCHUNK
NU = CH * NCHUNK


def _sc_body(src_fea, trg_fea, src_lab, trg_lab, out_s, out_c,
             lab_buf, fea_buf0, fea_buf1, sum_acc, cnt_acc, sem0, sem1):
    nc = 2
    wid = lax.axis_index("s") * nc + lax.axis_index("c")
    b = wid

    zeros = jnp.zeros((L,), jnp.float32)
    ones = jnp.full((L,), 1.0, jnp.float32)
    lane = lax.iota(jnp.int32, L)

    @plsc.parallel_loop(0, ACC_E // L, unroll=U)
    def _(i):
        sum_acc[pl.ds(i * L, L)] = zeros
        cnt_acc[pl.ds(i * L, L)] = zeros

    for tensor, (fea_hbm, lab_hbm) in enumerate(
            ((src_fea, src_lab), (trg_fea, trg_lab))):
        pltpu.sync_copy(lab_hbm.at[pl.ds(b * HW, HW)], lab_buf)

        sum_base_t = tensor * CH * CLS_PAD

        def _src_slice(u):
            ch = u // NCHUNK
            off = (u % NCHUNK) * P
            return fea_hbm.at[pl.ds((b * CH + ch) * HW + off, P)]

        def start_u(u, buf, sem):
            pltpu.async_copy(_src_slice(u), buf, sem)

        def wait_u(u, buf, sem):
            pltpu.make_async_copy(_src_slice(u), buf, sem).wait()

        def compute_u(u, buf):
            base = sum_base_t + (u // NCHUNK) * CLS_PAD
            lab_off = (u % NCHUNK) * P

            @plsc.parallel_loop(0, P // L, unroll=U)
            def _(p):
                o = p * L
                l16 = lab_buf[pl.ds(lab_off + o, L)]
                v16 = buf[pl.ds(o, L)]
                plsc.addupdate_scatter(
                    sum_acc, [(l16 + base) * L + lane], v16)

        start_u(0, fea_buf0, sem0)

        cnt_base = (tensor * B + b) * CLS_PAD

        @plsc.parallel_loop(0, HW // L, unroll=U)
        def _(p):
            l16 = lab_buf[pl.ds(p * L, L)]
            plsc.addupdate_scatter(
                cnt_acc, [(l16 + cnt_base) * L + lane], ones)

        def pipe_body(u2, c):
            ua = 2 * u2
            start_u(ua + 1, fea_buf1, sem1)
            wait_u(ua, fea_buf0, sem0)
            compute_u(ua, fea_buf0)

            @pl.when(ua + 2 < NU)
            def _():
                start_u(ua + 2, fea_buf0, sem0)

            wait_u(ua + 1, fea_buf1, sem1)
            compute_u(ua + 1, fea_buf1)
            return c
        lax.fori_loop(0, NU // 2, pipe_body, 0)

    pltpu.sync_copy(sum_acc, out_s.at[wid])
    pltpu.sync_copy(cnt_acc, out_c.at[wid])


_sc_seg = functools.partial(
    pl.kernel,
    mesh=plsc.VectorSubcoreMesh(core_axis_name="c", subcore_axis_name="s"),
    out_type=[
        jax.ShapeDtypeStruct((B, ACC_E), jnp.float32),
        jax.ShapeDtypeStruct((B, ACC_E), jnp.float32),
    ],
    scratch_types=[
        pltpu.VMEM((HW,), jnp.int32),
        pltpu.VMEM((P,), jnp.float32),
        pltpu.VMEM((P,), jnp.float32),
        pltpu.VMEM((ACC_E,), jnp.float32),
        pltpu.VMEM((ACC_E,), jnp.float32),
        pltpu.SemaphoreType.DMA,
        pltpu.SemaphoreType.DMA,
    ],
    compiler_params=pltpu.CompilerParams(needs_layout_passes=False),
)(_sc_body)


def _loss_body(s_ref, c_ref, o_ref):
    s = jnp.sum(s_ref[...], axis=0)
    n = jnp.sum(c_ref[...], axis=0)
    row = lax.broadcasted_iota(jnp.int32, (128, 8), 0)
    col = lax.broadcasted_iota(jnp.int32, (128, 8), 1)
    w = (row // L == col).astype(jnp.float32)
    s = jnp.dot(s, w, preferred_element_type=jnp.float32)
    n = jnp.dot(n, w, preferred_element_type=jnp.float32)
    ctr = s / ((n + 1e-8) * B)
    half = CH * CLS_PAD // 8
    d = ctr[:half, :] - ctr[half:, :]
    o_ref[0, 0] = jnp.sqrt(jnp.sum(d * d))


def kernel(src_fea, trg_fea, src_labels, trg_pseudo_labels):
    sf = src_fea.reshape(-1)
    tf = trg_fea.reshape(-1)
    sl = src_labels.reshape(-1).astype(jnp.int32)
    tl = trg_pseudo_labels.reshape(-1).astype(jnp.int32)

    out_s, out_c = _sc_seg(sf, tf, sl, tl)

    loss = pl.pallas_call(
        _loss_body,
        out_shape=jax.ShapeDtypeStruct((1, 1), jnp.float32),
        out_specs=pl.BlockSpec(memory_space=pltpu.SMEM),
    )(out_s.reshape(B, ACC // 8, 128), out_c.reshape(B, ACC // 8, 128))
    return loss[0, 0]
